# Initial kernel scaffold; baseline (speedup 1.0000x reference)
import jax
import jax.numpy as jnp
from jax.experimental import pallas as pl


def kernel(x, forward_shuffle_idx):
    raise NotImplementedError("write your pallas kernel here")


# speedup vs baseline: 2.0182x; 2.0182x over previous
import functools

import jax
import jax.numpy as jnp
from jax import lax
from jax.experimental import pallas as pl
from jax.experimental.pallas import tpu as pltpu
from jax.experimental.pallas import tpu_sc as plsc

_B, _T, _C = 16, 1024, 768
_NC, _NS = 2, 16
_NW = _NC * _NS
_ROWS_PER_W = _B * _T // _NW
_CHUNK = 64
_NCH = _ROWS_PER_W // _CHUNK
_LANES = 16


def _shuffle_body(x_hbm, idx_hbm, out_hbm, idx_v, buf0, buf1, sem0, sem1):
    b = lax.axis_index("s")
    half = lax.axis_index("c")
    out_base = (b * _NC + half) * _ROWS_PER_W

    pltpu.sync_copy(idx_hbm.at[pl.ds(half * _NCH, _NCH)], idx_v)
    boff = (b * _T).astype(jnp.int32)
    for j in range(_NCH):
        for i in range(_CHUNK // _LANES):
            sl = pl.ds(i * _LANES, _LANES)
            idx_v[j, sl] = idx_v[j, sl] + boff

    bufs = (buf0, buf1)
    sems = (sem0, sem1)
    copies = [None] * _NCH
    copies[0] = pltpu.async_copy(x_hbm.at[idx_v.at[0]], bufs[0], sems[0])
    for j in range(_NCH):
        if j + 1 < _NCH:
            copies[j + 1] = pltpu.async_copy(
                x_hbm.at[idx_v.at[j + 1]], bufs[(j + 1) % 2], sems[(j + 1) % 2])
        copies[j].wait()
        pltpu.sync_copy(bufs[j % 2],
                        out_hbm.at[pl.ds(out_base + j * _CHUNK, _CHUNK)])


_shuffle = functools.partial(
    pl.kernel,
    mesh=plsc.VectorSubcoreMesh(core_axis_name="c", subcore_axis_name="s"),
    out_type=jax.ShapeDtypeStruct((_B * _T, _C), jnp.float32),
    scratch_types=[
        pltpu.VMEM((_NCH, _CHUNK), jnp.int32),
        pltpu.VMEM((_CHUNK, _C), jnp.float32),
        pltpu.VMEM((_CHUNK, _C), jnp.float32),
        pltpu.SemaphoreType.DMA,
        pltpu.SemaphoreType.DMA,
    ],
)(_shuffle_body)


def kernel(x, forward_shuffle_idx):
    x2 = x.reshape(_B * _T, _C)
    idx2 = forward_shuffle_idx.reshape(_T // _CHUNK, _CHUNK)
    out = _shuffle(x2, idx2)
    return out.reshape(_B, _T, _C)
